# baseline (speedup 1.0000x reference)
import functools
import math

import jax
import jax.numpy as jnp
from jax.experimental import pallas as pl

N = 50000
E = 800000
EMBED = 64
HEADS = 8
DH = EMBED // HEADS

NBLK = 512
EBLK = 1024


def _ln(xv, g, b, eps=1e-5):
    m = jnp.mean(xv, axis=-1, keepdims=True)
    v = jnp.mean((xv - m) ** 2, axis=-1, keepdims=True)
    return (xv - m) * jax.lax.rsqrt(v + eps) * g + b


def _mm(a, w):
    return jax.lax.dot_general(a, w, (((1,), (0,)), ((), ())),
                               preferred_element_type=jnp.float32)


def _node_a_body(x_ref, rot_ref, bos_ref, bvec_ref,
                 w1, b1, g1, n1b, w2, b2, g2, n2b, w3, b3, g3, n3b,
                 ng, nb, wq, bq,
                 tbl_ref, center_ref):
    x0 = x_ref[:, 0:1]
    x1 = x_ref[:, 1:2]
    r = rot_ref[...]
    rx0 = x0 * r[:, 0:1] + x1 * r[:, 2:3]
    rx1 = x0 * r[:, 1:2] + x1 * r[:, 3:4]
    h = rx0 * w1[0:1, :] + rx1 * w1[1:2, :] + b1[...]
    h = jax.nn.relu(_ln(h, g1[...], n1b[...]))
    h = jax.nn.relu(_ln(_mm(h, w2[...]) + b2[...], g2[...], n2b[...]))
    center = _ln(_mm(h, w3[...]) + b3[...], g3[...], n3b[...])
    center = jnp.where(bos_ref[...] > 0.5, bvec_ref[...], center)
    center_ref[...] = center
    hn = _ln(center, ng[...], nb[...])
    q = _mm(hn, wq[...]) + bq[...]
    tbl_ref[:, 0:4] = r
    tbl_ref[:, 4:68] = q
    tbl_ref[:, 68:80] = jnp.zeros_like(tbl_ref[:, 68:80])


def _stage_a(x, rot4, bosf, bvec, p):
    grid = (pl.cdiv(N, NBLK),)
    row = lambda i: (i, 0)
    whole = lambda i: (0, 0)
    ins = [
        pl.BlockSpec((NBLK, 2), row),
        pl.BlockSpec((NBLK, 4), row),
        pl.BlockSpec((NBLK, 1), row),
        pl.BlockSpec((1, EMBED), whole),
    ]
    wspecs = []
    wvals = []
    for lin, lnp_ in (("ce_l1", "ce_n1"), ("ce_l2", "ce_n2"), ("ce_l3", "ce_n3")):
        w = p[lin]["W"]
        wvals += [w, p[lin]["b"].reshape(1, EMBED),
                  p[lnp_]["g"].reshape(1, EMBED), p[lnp_]["b"].reshape(1, EMBED)]
        wspecs += [pl.BlockSpec(w.shape, whole), pl.BlockSpec((1, EMBED), whole),
                   pl.BlockSpec((1, EMBED), whole), pl.BlockSpec((1, EMBED), whole)]
    wvals += [p["norm1"]["g"].reshape(1, EMBED), p["norm1"]["b"].reshape(1, EMBED),
              p["lin_q"]["W"], p["lin_q"]["b"].reshape(1, EMBED)]
    wspecs += [pl.BlockSpec((1, EMBED), whole), pl.BlockSpec((1, EMBED), whole),
               pl.BlockSpec((EMBED, EMBED), whole), pl.BlockSpec((1, EMBED), whole)]
    out_shapes = (jax.ShapeDtypeStruct((N, 80), jnp.float32),
                  jax.ShapeDtypeStruct((N, EMBED), jnp.float32))
    out_specs = (pl.BlockSpec((NBLK, 80), row), pl.BlockSpec((NBLK, EMBED), row))
    return pl.pallas_call(
        _node_a_body, grid=grid, in_specs=ins + wspecs,
        out_specs=out_specs, out_shape=out_shapes,
    )(x, rot4, bosf, bvec, *wvals)


def _edge_body(g_ref,
               wa1, ba1, ga1, na1, wa2, ba2,
               wb1, bb1, gb1, nb1, wb2, bb2,
               gan1, ban1, wal, bal, gan2, ban2,
               wk, bk, wv, bv, s8, t8,
               p0_ref, p1_ref):
    g = g_ref[...]
    xr0 = g[:, 0:1]
    xr1 = g[:, 1:2]
    er0 = g[:, 2:3]
    er1 = g[:, 3:4]
    q = g[:, 4:68]
    ha = xr0 * wa1[0:1, :] + xr1 * wa1[1:2, :] + ba1[...]
    ha = jax.nn.relu(_ln(ha, ga1[...], na1[...]))
    ha = _mm(ha, wa2[...]) + ba2[...]
    hb = er0 * wb1[0:1, :] + er1 * wb1[1:2, :] + bb1[...]
    hb = jax.nn.relu(_ln(hb, gb1[...], nb1[...]))
    hb = _mm(hb, wb2[...]) + bb2[...]
    s = ha + hb
    s = jax.nn.relu(_ln(s, gan1[...], ban1[...]))
    nbr = _ln(_mm(s, wal[...]) + bal[...], gan2[...], ban2[...])
    k = _mm(nbr, wk[...]) + bk[...]
    v = _mm(nbr, wv[...]) + bv[...]
    alpha = _mm(q * k, s8[...]) * (1.0 / math.sqrt(DH))
    ae = jnp.exp(alpha)
    aev = v * _mm(ae, t8[...])
    p0_ref[:, 0:32] = aev[:, 0:32]
    p0_ref[:, 32:40] = ae
    p1_ref[:, 0:32] = aev[:, 32:64]
    p1_ref[:, 32:40] = ae


def _stage_c(g, p):
    grid = (pl.cdiv(E, EBLK),)
    row = lambda i: (i, 0)
    whole = lambda i: (0, 0)
    s8 = jnp.zeros((EMBED, HEADS), jnp.float32)
    s8 = s8.at[jnp.arange(EMBED), jnp.arange(EMBED) // DH].set(1.0)
    t8 = s8.T
    wvals = []
    wspecs = []

    def add(w):
        wvals.append(w)
        wspecs.append(pl.BlockSpec(w.shape, whole))

    for lin in ("nb0_l1",):
        add(p[lin]["W"]); add(p[lin]["b"].reshape(1, EMBED))
    add(p["nb0_n1"]["g"].reshape(1, EMBED)); add(p["nb0_n1"]["b"].reshape(1, EMBED))
    add(p["nb0_l2"]["W"]); add(p["nb0_l2"]["b"].reshape(1, EMBED))
    add(p["nb1_l1"]["W"]); add(p["nb1_l1"]["b"].reshape(1, EMBED))
    add(p["nb1_n1"]["g"].reshape(1, EMBED)); add(p["nb1_n1"]["b"].reshape(1, EMBED))
    add(p["nb1_l2"]["W"]); add(p["nb1_l2"]["b"].reshape(1, EMBED))
    add(p["nb_an1"]["g"].reshape(1, EMBED)); add(p["nb_an1"]["b"].reshape(1, EMBED))
    add(p["nb_al"]["W"]); add(p["nb_al"]["b"].reshape(1, EMBED))
    add(p["nb_an2"]["g"].reshape(1, EMBED)); add(p["nb_an2"]["b"].reshape(1, EMBED))
    add(p["lin_k"]["W"]); add(p["lin_k"]["b"].reshape(1, EMBED))
    add(p["lin_v"]["W"]); add(p["lin_v"]["b"].reshape(1, EMBED))
    add(s8); add(t8)
    out_shapes = (jax.ShapeDtypeStruct((E, 40), jnp.float32),
                  jax.ShapeDtypeStruct((E, 40), jnp.float32))
    out_specs = (pl.BlockSpec((EBLK, 40), row), pl.BlockSpec((EBLK, 40), row))
    return pl.pallas_call(
        _edge_body, grid=grid,
        in_specs=[pl.BlockSpec((EBLK, 80), row)] + wspecs,
        out_specs=out_specs, out_shape=out_shapes,
    )(g, *wvals)


def _node_b_body(acc0_ref, acc1_ref, center_ref,
                 ng, nb, wih, bih, whh, bhh, ws, bs, wo, bo,
                 g2, b2n, wm1, bm1, wm2, bm2, t8,
                 out_ref):
    a0 = acc0_ref[...]
    a1 = acc1_ref[...]
    center = center_ref[...]
    s = jnp.concatenate([a0[:, 0:32], a1[:, 0:32]], axis=1)
    d = a0[:, 32:40]
    agg = s / (_mm(d, t8[...]) + 1e-16)
    h = _ln(center, ng[...], nb[...])
    gate = jax.nn.sigmoid(_mm(agg, wih[...]) + bih[...] + _mm(h, whh[...]) + bhh[...])
    upd = agg + gate * (_mm(h, ws[...]) + bs[...] - agg)
    center = center + _mm(upd, wo[...]) + bo[...]
    h2 = _ln(center, g2[...], b2n[...])
    ff = _mm(jax.nn.relu(_mm(h2, wm1[...]) + bm1[...]), wm2[...]) + bm2[...]
    out_ref[...] = center + ff


def _stage_e(acc0, acc1, center, p):
    grid = (pl.cdiv(N, NBLK),)
    row = lambda i: (i, 0)
    whole = lambda i: (0, 0)
    t8 = jnp.zeros((HEADS, EMBED), jnp.float32)
    t8 = t8.at[jnp.arange(EMBED) // DH, jnp.arange(EMBED)].set(1.0)
    wvals = []
    wspecs = []

    def add(w):
        wvals.append(w)
        wspecs.append(pl.BlockSpec(w.shape, whole))

    add(p["norm1"]["g"].reshape(1, EMBED)); add(p["norm1"]["b"].reshape(1, EMBED))
    add(p["lin_ih"]["W"]); add(p["lin_ih"]["b"].reshape(1, EMBED))
    add(p["lin_hh"]["W"]); add(p["lin_hh"]["b"].reshape(1, EMBED))
    add(p["lin_self"]["W"]); add(p["lin_self"]["b"].reshape(1, EMBED))
    add(p["out_proj"]["W"]); add(p["out_proj"]["b"].reshape(1, EMBED))
    add(p["norm2"]["g"].reshape(1, EMBED)); add(p["norm2"]["b"].reshape(1, EMBED))
    add(p["mlp_l1"]["W"]); add(p["mlp_l1"]["b"].reshape(1, EMBED * 4))
    add(p["mlp_l2"]["W"]); add(p["mlp_l2"]["b"].reshape(1, EMBED))
    add(t8)
    return pl.pallas_call(
        _node_b_body, grid=grid,
        in_specs=[pl.BlockSpec((NBLK, 40), row), pl.BlockSpec((NBLK, 40), row),
                  pl.BlockSpec((NBLK, EMBED), row)] + wspecs,
        out_specs=pl.BlockSpec((NBLK, EMBED), row),
        out_shape=jax.ShapeDtypeStruct((N, EMBED), jnp.float32),
    )(acc0, acc1, center, *wvals)


def kernel(x, t, edge_index, edge_attr, bos_mask, rotate_mat, params):
    src = edge_index[0]
    dst = edge_index[1]
    rot4 = rotate_mat.reshape(N, 4)
    bosf = bos_mask.astype(jnp.float32).reshape(N, 1)
    bvec = jax.lax.dynamic_slice_in_dim(params["bos_token"], t, 1, 0)

    tbl, center = _stage_a(x, rot4, bosf, bvec, params)

    xj = x[src]
    rg = tbl[dst, 0:4]
    qg = tbl[dst, 4:68]
    xr0 = xj[:, 0:1] * rg[:, 0:1] + xj[:, 1:2] * rg[:, 2:3]
    xr1 = xj[:, 0:1] * rg[:, 1:2] + xj[:, 1:2] * rg[:, 3:4]
    er0 = edge_attr[:, 0:1] * rg[:, 0:1] + edge_attr[:, 1:2] * rg[:, 2:3]
    er1 = edge_attr[:, 0:1] * rg[:, 1:2] + edge_attr[:, 1:2] * rg[:, 3:4]
    g = jnp.concatenate([xr0, xr1, er0, er1, qg,
                         jnp.zeros((E, 12), jnp.float32)], axis=1)

    p0, p1 = _stage_c(g, params)

    acc0 = jax.ops.segment_sum(p0, dst, num_segments=N)
    acc1 = jax.ops.segment_sum(p1, dst, num_segments=N)

    return _stage_e(acc0, acc1, center, params)


# speedup vs baseline: 34.2517x; 34.2517x over previous
import functools
import math

import jax
import jax.numpy as jnp
from jax import lax
from jax.experimental import pallas as pl
from jax.experimental.pallas import tpu as pltpu
from jax.experimental.pallas import tpu_sc as plsc

N = 50000
E = 800000
EMBED = 64
HEADS = 8
DH = EMBED // HEADS

NBLK = 512
EBLK = 1024

NC = 2
NS = 16
NW = NC * NS
IB = 128
EPAD = 819200
ROWS = EPAD // IB
RPW = ROWS // NW
CHR = 4
CH = CHR * IB
NCH = RPW // CHR
CHRD = 8
CHD = CHRD * IB
RPT = ROWS // NS
NCHD = RPT // CHRD
XROWS = 50048
ACC_ROWS = 50048
ACC_SLICE = ACC_ROWS // NS

@functools.cache
def _sc_mesh():
    return plsc.VectorSubcoreMesh(core_axis_name="c", subcore_axis_name="s",
                                  num_cores=NC, num_subcores=NS)


def _ln(xv, g, b, eps=1e-5):
    m = jnp.mean(xv, axis=-1, keepdims=True)
    v = jnp.mean((xv - m) ** 2, axis=-1, keepdims=True)
    return (xv - m) * jax.lax.rsqrt(v + eps) * g + b


def _mm(a, w):
    return jax.lax.dot_general(a, w, (((1,), (0,)), ((), ())),
                               preferred_element_type=jnp.float32)


def _node_a_body(x_ref, rot_ref, bos_ref, bvec_ref,
                 w1, b1, g1, n1b, w2, b2, g2, n2b, w3, b3, g3, n3b,
                 ng, nb, wq, bq,
                 tbl_ref, center_ref):
    x0 = x_ref[:, 0:1]
    x1 = x_ref[:, 1:2]
    r = rot_ref[...]
    rx0 = x0 * r[:, 0:1] + x1 * r[:, 2:3]
    rx1 = x0 * r[:, 1:2] + x1 * r[:, 3:4]
    h = rx0 * w1[0:1, :] + rx1 * w1[1:2, :] + b1[...]
    h = jax.nn.relu(_ln(h, g1[...], n1b[...]))
    h = jax.nn.relu(_ln(_mm(h, w2[...]) + b2[...], g2[...], n2b[...]))
    center = _ln(_mm(h, w3[...]) + b3[...], g3[...], n3b[...])
    center = jnp.where(bos_ref[...] > 0.5, bvec_ref[...], center)
    center_ref[...] = center
    hn = _ln(center, ng[...], nb[...])
    tbl_ref[...] = _mm(hn, wq[...]) + bq[...]


def _stage_a(x, rot4, bosf, bvec, p):
    grid = (pl.cdiv(N, NBLK),)
    row = lambda i: (i, 0)
    whole = lambda i: (0, 0)
    ins = [
        pl.BlockSpec((NBLK, 2), row),
        pl.BlockSpec((NBLK, 4), row),
        pl.BlockSpec((NBLK, 1), row),
        pl.BlockSpec((1, EMBED), whole),
    ]
    wspecs = []
    wvals = []
    for lin, lnp_ in (("ce_l1", "ce_n1"), ("ce_l2", "ce_n2"), ("ce_l3", "ce_n3")):
        w = p[lin]["W"]
        wvals += [w, p[lin]["b"].reshape(1, EMBED),
                  p[lnp_]["g"].reshape(1, EMBED), p[lnp_]["b"].reshape(1, EMBED)]
        wspecs += [pl.BlockSpec(w.shape, whole), pl.BlockSpec((1, EMBED), whole),
                   pl.BlockSpec((1, EMBED), whole), pl.BlockSpec((1, EMBED), whole)]
    wvals += [p["norm1"]["g"].reshape(1, EMBED), p["norm1"]["b"].reshape(1, EMBED),
              p["lin_q"]["W"], p["lin_q"]["b"].reshape(1, EMBED)]
    wspecs += [pl.BlockSpec((1, EMBED), whole), pl.BlockSpec((1, EMBED), whole),
               pl.BlockSpec((EMBED, EMBED), whole), pl.BlockSpec((1, EMBED), whole)]
    out_shapes = (jax.ShapeDtypeStruct((N, EMBED), jnp.float32),
                  jax.ShapeDtypeStruct((N, EMBED), jnp.float32))
    out_specs = (pl.BlockSpec((NBLK, EMBED), row), pl.BlockSpec((NBLK, EMBED), row))
    return pl.pallas_call(
        _node_a_body, grid=grid, in_specs=ins + wspecs,
        out_specs=out_specs, out_shape=out_shapes,
    )(x, rot4, bosf, bvec, *wvals)


def _gather_body(dst2d, src2d, qtbl, rtbl, xtbl, qg_out, rg_out, xg_out,
                 dbuf, sbuf, qbuf, rbuf, xbuf, qsem, rsem, xsem):
    wid = lax.axis_index("s") * NC + lax.axis_index("c")

    def chunk(i, carry):
        row0 = wid * RPW + i * CHR
        pltpu.sync_copy(dst2d.at[pl.ds(row0, CHR)], dbuf)
        pltpu.sync_copy(src2d.at[pl.ds(row0, CHR)], sbuf)
        for j in range(CHR):
            pltpu.async_copy(qtbl.at[dbuf.at[j]], qbuf.at[pl.ds(j * IB, IB)], qsem)
            pltpu.async_copy(rtbl.at[dbuf.at[j]], rbuf.at[pl.ds(j * IB, IB)], rsem)
            pltpu.async_copy(xtbl.at[sbuf.at[j]], xbuf.at[pl.ds(j * IB, IB)], xsem)
        for j in range(CHR):
            pltpu.make_async_copy(qtbl.at[dbuf.at[j]],
                                  qbuf.at[pl.ds(j * IB, IB)], qsem).wait()
            pltpu.make_async_copy(rtbl.at[dbuf.at[j]],
                                  rbuf.at[pl.ds(j * IB, IB)], rsem).wait()
            pltpu.make_async_copy(xtbl.at[sbuf.at[j]],
                                  xbuf.at[pl.ds(j * IB, IB)], xsem).wait()
        base = row0 * IB
        pltpu.sync_copy(qbuf, qg_out.at[pl.ds(base, CH)])
        pltpu.sync_copy(rbuf, rg_out.at[pl.ds(base, CH)])
        pltpu.sync_copy(xbuf, xg_out.at[pl.ds(base, CH)])
        return carry

    lax.fori_loop(0, NCH, chunk, 0)


@functools.cache
def _gather_call():
    return pl.kernel(
        _gather_body,
        out_type=(jax.ShapeDtypeStruct((EPAD, 64), jnp.float32),
                  jax.ShapeDtypeStruct((EPAD, 16), jnp.float32),
                  jax.ShapeDtypeStruct((EPAD, 8), jnp.float32)),
        mesh=_sc_mesh(),
        scratch_types=[
            pltpu.VMEM((CHR, IB), jnp.int32),
            pltpu.VMEM((CHR, IB), jnp.int32),
            pltpu.VMEM((CH, 64), jnp.float32),
            pltpu.VMEM((CH, 16), jnp.float32),
            pltpu.VMEM((CH, 8), jnp.float32),
            pltpu.SemaphoreType.DMA,
            pltpu.SemaphoreType.DMA,
            pltpu.SemaphoreType.DMA,
        ],
        compiler_params=pltpu.CompilerParams(use_tc_tiling_on_sc=False),
    )


def _edge_body(q_ref, r_ref, xg_ref, ea_ref,
               wa1, ba1, ga1, na1, wa2, ba2,
               wb1, bb1, gb1, nb1, wb2, bb2,
               gan1, ban1, wal, bal, gan2, ban2,
               wk, bk, wv, bv, s8, t8,
               p0_ref, p1_ref, p2_ref, p3_ref, p4_ref):
    rr = r_ref[...]
    r00 = rr[:, 0:1]
    r01 = rr[:, 1:2]
    r10 = rr[:, 2:3]
    r11 = rr[:, 3:4]
    x0 = xg_ref[:, 0:1]
    x1 = xg_ref[:, 1:2]
    ea0 = ea_ref[:, 0:1]
    ea1 = ea_ref[:, 1:2]
    xr0 = x0 * r00 + x1 * r10
    xr1 = x0 * r01 + x1 * r11
    er0 = ea0 * r00 + ea1 * r10
    er1 = ea0 * r01 + ea1 * r11
    q = q_ref[...]
    ha = xr0 * wa1[0:1, :] + xr1 * wa1[1:2, :] + ba1[...]
    ha = jax.nn.relu(_ln(ha, ga1[...], na1[...]))
    ha = _mm(ha, wa2[...]) + ba2[...]
    hb = er0 * wb1[0:1, :] + er1 * wb1[1:2, :] + bb1[...]
    hb = jax.nn.relu(_ln(hb, gb1[...], nb1[...]))
    hb = _mm(hb, wb2[...]) + bb2[...]
    s = ha + hb
    s = jax.nn.relu(_ln(s, gan1[...], ban1[...]))
    nbr = _ln(_mm(s, wal[...]) + bal[...], gan2[...], ban2[...])
    k = _mm(nbr, wk[...]) + bk[...]
    v = _mm(nbr, wv[...]) + bv[...]
    alpha = _mm(q * k, s8[...]) * (1.0 / math.sqrt(DH))
    ae = jnp.exp(alpha)
    aev = v * _mm(ae, t8[...])
    p0_ref[...] = aev[:, 0:16]
    p1_ref[...] = aev[:, 16:32]
    p2_ref[...] = aev[:, 32:48]
    p3_ref[...] = aev[:, 48:64]
    p4_ref[:, 0:8] = ae
    p4_ref[:, 8:16] = jnp.zeros_like(p4_ref[:, 8:16])


def _stage_c(qg, rg, xg, ea, p):
    grid = (EPAD // EBLK,)
    row = lambda i: (i, 0)
    whole = lambda i: (0, 0)
    s8 = jnp.zeros((EMBED, HEADS), jnp.float32)
    s8 = s8.at[jnp.arange(EMBED), jnp.arange(EMBED) // DH].set(1.0)
    t8 = s8.T
    wvals = []
    wspecs = []

    def add(w):
        wvals.append(w)
        wspecs.append(pl.BlockSpec(w.shape, whole))

    add(p["nb0_l1"]["W"]); add(p["nb0_l1"]["b"].reshape(1, EMBED))
    add(p["nb0_n1"]["g"].reshape(1, EMBED)); add(p["nb0_n1"]["b"].reshape(1, EMBED))
    add(p["nb0_l2"]["W"]); add(p["nb0_l2"]["b"].reshape(1, EMBED))
    add(p["nb1_l1"]["W"]); add(p["nb1_l1"]["b"].reshape(1, EMBED))
    add(p["nb1_n1"]["g"].reshape(1, EMBED)); add(p["nb1_n1"]["b"].reshape(1, EMBED))
    add(p["nb1_l2"]["W"]); add(p["nb1_l2"]["b"].reshape(1, EMBED))
    add(p["nb_an1"]["g"].reshape(1, EMBED)); add(p["nb_an1"]["b"].reshape(1, EMBED))
    add(p["nb_al"]["W"]); add(p["nb_al"]["b"].reshape(1, EMBED))
    add(p["nb_an2"]["g"].reshape(1, EMBED)); add(p["nb_an2"]["b"].reshape(1, EMBED))
    add(p["lin_k"]["W"]); add(p["lin_k"]["b"].reshape(1, EMBED))
    add(p["lin_v"]["W"]); add(p["lin_v"]["b"].reshape(1, EMBED))
    add(s8); add(t8)
    out_shapes = tuple(jax.ShapeDtypeStruct((EPAD, 16), jnp.float32)
                       for _ in range(5))
    out_specs = tuple(pl.BlockSpec((EBLK, 16), row) for _ in range(5))
    return pl.pallas_call(
        _edge_body, grid=grid,
        in_specs=[pl.BlockSpec((EBLK, 64), row), pl.BlockSpec((EBLK, 16), row),
                  pl.BlockSpec((EBLK, 8), row),
                  pl.BlockSpec((EBLK, 2), row)] + wspecs,
        out_specs=out_specs, out_shape=out_shapes,
    )(qg, rg, xg, ea, *wvals)


def _scatter_body(dstf, p0, p1, p2, p3, p4, zeros_hbm,
                  out0, out1, out2, out3, out4,
                  ib0, ib1, ib2, ib3, ib4, ib5, ib6, ib7,
                  pb0, pb1, pb2, pb3, pb4, pb5, pb6, pb7, acc):
    c = lax.axis_index("c")
    s = lax.axis_index("s")
    base = s * ACC_SLICE
    ibufs = (ib0, ib1, ib2, ib3, ib4, ib5, ib6, ib7)
    pbufs = (pb0, pb1, pb2, pb3, pb4, pb5, pb6, pb7)

    def one_pass(p_ref, out_ref):
        pltpu.sync_copy(zeros_hbm, acc.at[pl.ds(base, ACC_SLICE)])
        plsc.subcore_barrier()

        def chunk(i, carry):
            e0 = (s * RPT + i * CHRD) * IB
            for j in range(CHRD):
                pltpu.sync_copy(dstf.at[pl.ds(e0 + j * IB, IB)], ibufs[j])
                pltpu.sync_copy(p_ref.at[pl.ds(e0 + j * IB, IB)], pbufs[j])
            for j in range(CHRD):
                pltpu.sync_copy(pbufs[j], acc.at[ibufs[j]], add=True)
            return carry

        lax.fori_loop(0, NCHD, chunk, 0)
        plsc.subcore_barrier()
        pltpu.sync_copy(acc.at[pl.ds(base, ACC_SLICE)],
                        out_ref.at[pl.ds(base, ACC_SLICE)])
        plsc.subcore_barrier()

    @pl.when(c == 0)
    def _():
        one_pass(p0, out0)
        one_pass(p1, out1)
        one_pass(p4, out4)

    @pl.when(c == 1)
    def _():
        one_pass(p2, out2)
        one_pass(p3, out3)


@functools.cache
def _scatter_call():
    return pl.kernel(
        _scatter_body,
        out_type=tuple(jax.ShapeDtypeStruct((ACC_ROWS, 16), jnp.float32)
                       for _ in range(5)),
        mesh=_sc_mesh(),
        scratch_types=[pltpu.VMEM((IB,), jnp.int32)] * 8
        + [pltpu.VMEM((IB, 16), jnp.float32)] * 8
        + [pltpu.VMEM_SHARED((ACC_ROWS, 16), jnp.float32)],
        compiler_params=pltpu.CompilerParams(use_tc_tiling_on_sc=False),
    )


def _node_b_body(acc0_ref, acc1_ref, acc2_ref, acc3_ref, acc4_ref, center_ref,
                 ng, nb, wih, bih, whh, bhh, ws, bs, wo, bo,
                 g2, b2n, wm1, bm1, wm2, bm2, t8,
                 out_ref):
    center = center_ref[...]
    s = jnp.concatenate([acc0_ref[...], acc1_ref[...], acc2_ref[...],
                         acc3_ref[...]], axis=1)
    d = acc4_ref[:, 0:8]
    agg = s / (_mm(d, t8[...]) + 1e-16)
    h = _ln(center, ng[...], nb[...])
    gate = jax.nn.sigmoid(_mm(agg, wih[...]) + bih[...] + _mm(h, whh[...]) + bhh[...])
    upd = agg + gate * (_mm(h, ws[...]) + bs[...] - agg)
    center = center + _mm(upd, wo[...]) + bo[...]
    h2 = _ln(center, g2[...], b2n[...])
    ff = _mm(jax.nn.relu(_mm(h2, wm1[...]) + bm1[...]), wm2[...]) + bm2[...]
    out_ref[...] = center + ff


def _stage_e(accs, center, p):
    grid = (pl.cdiv(N, NBLK),)
    row = lambda i: (i, 0)
    whole = lambda i: (0, 0)
    t8 = jnp.zeros((HEADS, EMBED), jnp.float32)
    t8 = t8.at[jnp.arange(EMBED) // DH, jnp.arange(EMBED)].set(1.0)
    wvals = []
    wspecs = []

    def add(w):
        wvals.append(w)
        wspecs.append(pl.BlockSpec(w.shape, whole))

    add(p["norm1"]["g"].reshape(1, EMBED)); add(p["norm1"]["b"].reshape(1, EMBED))
    add(p["lin_ih"]["W"]); add(p["lin_ih"]["b"].reshape(1, EMBED))
    add(p["lin_hh"]["W"]); add(p["lin_hh"]["b"].reshape(1, EMBED))
    add(p["lin_self"]["W"]); add(p["lin_self"]["b"].reshape(1, EMBED))
    add(p["out_proj"]["W"]); add(p["out_proj"]["b"].reshape(1, EMBED))
    add(p["norm2"]["g"].reshape(1, EMBED)); add(p["norm2"]["b"].reshape(1, EMBED))
    add(p["mlp_l1"]["W"]); add(p["mlp_l1"]["b"].reshape(1, EMBED * 4))
    add(p["mlp_l2"]["W"]); add(p["mlp_l2"]["b"].reshape(1, EMBED))
    add(t8)
    return pl.pallas_call(
        _node_b_body, grid=grid,
        in_specs=[pl.BlockSpec((NBLK, 16), row)] * 5
        + [pl.BlockSpec((NBLK, EMBED), row)] + wspecs,
        out_specs=pl.BlockSpec((NBLK, EMBED), row),
        out_shape=jax.ShapeDtypeStruct((N, EMBED), jnp.float32),
    )(*accs, center, *wvals)


def kernel(x, t, edge_index, edge_attr, bos_mask, rotate_mat, params):
    src = edge_index[0]
    dst = edge_index[1]
    rot4 = rotate_mat.reshape(N, 4)
    bosf = bos_mask.astype(jnp.float32).reshape(N, 1)
    bvec = jax.lax.dynamic_slice_in_dim(params["bos_token"], t, 1, 0)

    dstb = jnp.pad(dst, (0, EPAD - E)).reshape(ROWS, IB)
    dstd = jnp.pad(dst, (0, EPAD - E), constant_values=N)
    srcb = jnp.pad(src, (0, EPAD - E)).reshape(ROWS, IB)
    eap = jnp.pad(edge_attr, ((0, EPAD - E), (0, 0)))
    rtbl = jnp.pad(rot4, ((0, 0), (0, 12)))
    xtbl = jnp.pad(x, ((0, 0), (0, 6)))
    zeros_acc = jnp.zeros((ACC_SLICE, 16), jnp.float32)

    qtbl, center = _stage_a(x, rot4, bosf, bvec, params)
    qg, rg, xg = _gather_call()(dstb, srcb, qtbl, rtbl, xtbl)
    ps = _stage_c(qg, rg, xg, eap, params)
    accs = _scatter_call()(dstd, *ps, zeros_acc)
    return _stage_e(accs, center, params)
